# baseline (speedup 1.0000x reference)
import functools

import jax
import jax.numpy as jnp
from jax.experimental import pallas as pl

_V = 1.0


def _layer1_kernel(adj_ref, x_ref, w1_ref, w2_ref, snext_ref, adjb_ref):
    adjb = adj_ref[...].astype(jnp.bfloat16)
    adjb_ref[...] = adjb
    s1 = jnp.dot(x_ref[...], w1_ref[...],
                 preferred_element_type=jnp.float32).astype(jnp.bfloat16)
    h = jnp.dot(adjb, s1, preferred_element_type=jnp.float32)
    h = jnp.maximum(h, 0.0)
    snext_ref[...] = jnp.dot(h, w2_ref[...],
                             preferred_element_type=jnp.float32
                             ).astype(snext_ref.dtype)


def _layer_s_kernel(adj_ref, s_ref, w_ref, snext_ref):
    h = jnp.dot(adj_ref[...], s_ref[...], preferred_element_type=jnp.float32)
    h = jnp.maximum(h, 0.0)
    snext_ref[...] = jnp.dot(h, w_ref[...],
                             preferred_element_type=jnp.float32
                             ).astype(snext_ref.dtype)


def _layer_zs_kernel(adj_ref, s_ref, w_ref, z_ref, snext_ref):
    z = jnp.dot(adj_ref[...], s_ref[...], preferred_element_type=jnp.float32)
    z_ref[...] = z
    snext_ref[...] = jnp.dot(z, w_ref[...],
                             preferred_element_type=jnp.float32
                             ).astype(snext_ref.dtype)


def _layer_split_kernel(adj_ref, s_ref, w_ref, zc_ref, snext_ref, *, split):
    o = jnp.dot(adj_ref[...], s_ref[...], preferred_element_type=jnp.float32)
    o = jnp.maximum(o, 0.0)
    zc_ref[...] = o[:, :split]
    snext_ref[...] = jnp.dot(o[:, split:], w_ref[...],
                             preferred_element_type=jnp.float32
                             ).astype(snext_ref.dtype)


def _final_kernel(adj_ref, s_ref, zb_ref, z_ref, c_ref,
                  zhat_ref, zadj_ref, q_ref, *, k):
    o = jnp.dot(adj_ref[...], s_ref[...], preferred_element_type=jnp.float32)
    zhat_ref[...] = jnp.maximum(o, 0.0)
    zb = zb_ref[...]
    logits = jax.lax.dot_general(
        zb, z_ref[...], (((1,), (1,)), ((), ())),
        preferred_element_type=jnp.float32)
    zadj_ref[...] = jax.nn.sigmoid(logits)
    c = c_ref[...]
    cross = jax.lax.dot_general(
        zb, c, (((1,), (1,)), ((), ())),
        preferred_element_type=jnp.float32)
    d2 = (jnp.sum(zb * zb, axis=1, keepdims=True)
          + jnp.sum(c * c, axis=1)[None, :] - 2.0 * cross)
    qn = 1.0 / (1.0 + d2 / _V)
    qn = qn ** ((_V + 1.0) / 2.0)
    qn = qn[:, :k]
    q_ref[...] = qn / jnp.sum(qn, axis=1, keepdims=True)


def _block_m(n, target):
    for bm in (2000, 1000, 400, 200, 8):
        if bm <= target and n % bm == 0:
            return bm
    return n


def _gnn_pass(kernel_fn, adj, s, w, outs, bm_target=400):
    n = adj.shape[0]
    bm = _block_m(n, bm_target)
    grid = (n // bm,)
    in_specs = [
        pl.BlockSpec((bm, n), lambda i: (i, 0)),
        pl.BlockSpec(s.shape, lambda i: (0, 0)),
    ]
    args = [adj, s]
    if w is not None:
        in_specs.append(pl.BlockSpec(w.shape, lambda i: (0, 0)))
        args.append(w)
    out_shape = [jax.ShapeDtypeStruct((n, fw), dt) for fw, dt in outs]
    out_specs = [pl.BlockSpec((bm, fw), lambda i: (i, 0)) for fw, _ in outs]
    res = pl.pallas_call(
        kernel_fn, grid=grid, in_specs=in_specs, out_specs=out_specs,
        out_shape=out_shape)(*args)
    return res if len(res) > 1 else res[0]


def kernel(x, adj, W1, W2, W3, Wc, W4, W5, W6, cluster_layer):
    n = adj.shape[0]
    k, e = cluster_layer.shape
    kpad = max(8, -(-k // 8) * 8)
    c_pad = jnp.zeros((kpad, e), jnp.float32).at[:k].set(cluster_layer)
    f32, bf16 = jnp.float32, jnp.bfloat16

    bm1 = _block_m(n, 400)
    s2, adjb = pl.pallas_call(
        _layer1_kernel,
        grid=(n // bm1,),
        in_specs=[pl.BlockSpec((bm1, n), lambda i: (i, 0)),
                  pl.BlockSpec(x.shape, lambda i: (0, 0)),
                  pl.BlockSpec(W1.shape, lambda i: (0, 0)),
                  pl.BlockSpec(W2.shape, lambda i: (0, 0))],
        out_specs=[pl.BlockSpec((bm1, W2.shape[1]), lambda i: (i, 0)),
                   pl.BlockSpec((bm1, n), lambda i: (i, 0))],
        out_shape=[jax.ShapeDtypeStruct((n, W2.shape[1]), bf16),
                   jax.ShapeDtypeStruct((n, n), bf16)])(adj, x, W1, W2)
    s3 = _gnn_pass(_layer_s_kernel, adjb, s2, W3, [(W3.shape[1], bf16)])
    w_cat = jnp.concatenate([Wc, W4], axis=1)
    z, s4 = _gnn_pass(_layer_zs_kernel, adjb, s3, w_cat,
                      [(e, f32), (w_cat.shape[1], bf16)])
    z_cluster, s5 = _gnn_pass(
        functools.partial(_layer_split_kernel, split=k),
        adjb, s4, W5, [(k, f32), (W5.shape[1], bf16)])
    s6 = _gnn_pass(_layer_s_kernel, adjb, s5, W6, [(W6.shape[1], bf16)])

    bm = _block_m(n, 400)
    z_hat, z_adj, q = pl.pallas_call(
        functools.partial(_final_kernel, k=k),
        grid=(n // bm,),
        in_specs=[pl.BlockSpec((bm, n), lambda i: (i, 0)),
                  pl.BlockSpec(s6.shape, lambda i: (0, 0)),
                  pl.BlockSpec((bm, e), lambda i: (i, 0)),
                  pl.BlockSpec((n, e), lambda i: (0, 0)),
                  pl.BlockSpec((kpad, e), lambda i: (0, 0))],
        out_specs=[pl.BlockSpec((bm, W6.shape[1]), lambda i: (i, 0)),
                   pl.BlockSpec((bm, n), lambda i: (i, 0)),
                   pl.BlockSpec((bm, k), lambda i: (i, 0))],
        out_shape=[jax.ShapeDtypeStruct((n, W6.shape[1]), f32),
                   jax.ShapeDtypeStruct((n, n), f32),
                   jax.ShapeDtypeStruct((n, k), f32)])(adjb, s6, z, z, c_pad)

    return (z_hat, z_adj, z, z_cluster, q)


# speedup vs baseline: 1.0388x; 1.0388x over previous
import functools

import jax
import jax.numpy as jnp
from jax.experimental import pallas as pl
from jax.experimental.pallas import tpu as pltpu

_V = 1.0
_PAR = pltpu.CompilerParams(dimension_semantics=("parallel",))


def _layer1_kernel(adj_ref, x_ref, w1_ref, w2_ref, snext_ref, adjb_ref):
    adjb = adj_ref[...].astype(jnp.bfloat16)
    adjb_ref[...] = adjb
    s1 = jnp.dot(x_ref[...], w1_ref[...],
                 preferred_element_type=jnp.float32).astype(jnp.bfloat16)
    h = jnp.dot(adjb, s1, preferred_element_type=jnp.float32)
    h = jnp.maximum(h, 0.0)
    snext_ref[...] = jnp.dot(h, w2_ref[...],
                             preferred_element_type=jnp.float32
                             ).astype(snext_ref.dtype)


def _layer_s_kernel(adj_ref, s_ref, w_ref, snext_ref):
    h = jnp.dot(adj_ref[...], s_ref[...], preferred_element_type=jnp.float32)
    h = jnp.maximum(h, 0.0)
    snext_ref[...] = jnp.dot(h, w_ref[...],
                             preferred_element_type=jnp.float32
                             ).astype(snext_ref.dtype)


def _layer_zs_kernel(adj_ref, s_ref, w_ref, z_ref, snext_ref):
    z = jnp.dot(adj_ref[...], s_ref[...], preferred_element_type=jnp.float32)
    z_ref[...] = z
    snext_ref[...] = jnp.dot(z, w_ref[...],
                             preferred_element_type=jnp.float32
                             ).astype(snext_ref.dtype)


def _layer_split_kernel(adj_ref, s_ref, w_ref, zc_ref, snext_ref, *, split):
    o = jnp.dot(adj_ref[...], s_ref[...], preferred_element_type=jnp.float32)
    o = jnp.maximum(o, 0.0)
    zc_ref[...] = o[:, :split]
    snext_ref[...] = jnp.dot(o[:, split:], w_ref[...],
                             preferred_element_type=jnp.float32
                             ).astype(snext_ref.dtype)


def _final_kernel(adj_ref, s_ref, zb_ref, z_ref, c_ref,
                  zhat_ref, zadj_ref, q_ref, *, k):
    o = jnp.dot(adj_ref[...], s_ref[...], preferred_element_type=jnp.float32)
    zhat_ref[...] = jnp.maximum(o, 0.0)
    zb = zb_ref[...]
    logits = jax.lax.dot_general(
        zb, z_ref[...], (((1,), (1,)), ((), ())),
        preferred_element_type=jnp.float32)
    zadj_ref[...] = jax.nn.sigmoid(logits)
    c = c_ref[...]
    cross = jax.lax.dot_general(
        zb, c, (((1,), (1,)), ((), ())),
        preferred_element_type=jnp.float32)
    d2 = (jnp.sum(zb * zb, axis=1, keepdims=True)
          + jnp.sum(c * c, axis=1)[None, :] - 2.0 * cross)
    qn = 1.0 / (1.0 + d2 / _V)
    qn = qn ** ((_V + 1.0) / 2.0)
    qn = qn[:, :k]
    q_ref[...] = qn / jnp.sum(qn, axis=1, keepdims=True)


def _block_m(n, target):
    for bm in (2000, 1000, 400, 200, 8):
        if bm <= target and n % bm == 0:
            return bm
    return n


def _gnn_pass(kernel_fn, adj, s, w, outs, bm_target=1000):
    n = adj.shape[0]
    bm = _block_m(n, bm_target)
    grid = (n // bm,)
    in_specs = [
        pl.BlockSpec((bm, n), lambda i: (i, 0)),
        pl.BlockSpec(s.shape, lambda i: (0, 0)),
    ]
    args = [adj, s]
    if w is not None:
        in_specs.append(pl.BlockSpec(w.shape, lambda i: (0, 0)))
        args.append(w)
    out_shape = [jax.ShapeDtypeStruct((n, fw), dt) for fw, dt in outs]
    out_specs = [pl.BlockSpec((bm, fw), lambda i: (i, 0)) for fw, _ in outs]
    res = pl.pallas_call(
        kernel_fn, grid=grid, in_specs=in_specs, out_specs=out_specs,
        out_shape=out_shape, compiler_params=_PAR)(*args)
    return res if len(res) > 1 else res[0]


def kernel(x, adj, W1, W2, W3, Wc, W4, W5, W6, cluster_layer):
    n = adj.shape[0]
    k, e = cluster_layer.shape
    kpad = max(8, -(-k // 8) * 8)
    c_pad = jnp.zeros((kpad, e), jnp.float32).at[:k].set(cluster_layer)
    f32, bf16 = jnp.float32, jnp.bfloat16

    bm1 = _block_m(n, 400)
    s2, adjb = pl.pallas_call(
        _layer1_kernel,
        grid=(n // bm1,),
        in_specs=[pl.BlockSpec((bm1, n), lambda i: (i, 0)),
                  pl.BlockSpec(x.shape, lambda i: (0, 0)),
                  pl.BlockSpec(W1.shape, lambda i: (0, 0)),
                  pl.BlockSpec(W2.shape, lambda i: (0, 0))],
        out_specs=[pl.BlockSpec((bm1, W2.shape[1]), lambda i: (i, 0)),
                   pl.BlockSpec((bm1, n), lambda i: (i, 0))],
        out_shape=[jax.ShapeDtypeStruct((n, W2.shape[1]), bf16),
                   jax.ShapeDtypeStruct((n, n), bf16)],
        compiler_params=_PAR)(adj, x, W1, W2)
    s3 = _gnn_pass(_layer_s_kernel, adjb, s2, W3, [(W3.shape[1], bf16)])
    w_cat = jnp.concatenate([Wc, W4], axis=1)
    z, s4 = _gnn_pass(_layer_zs_kernel, adjb, s3, w_cat,
                      [(e, f32), (w_cat.shape[1], bf16)])
    z_cluster, s5 = _gnn_pass(
        functools.partial(_layer_split_kernel, split=k),
        adjb, s4, W5, [(k, f32), (W5.shape[1], bf16)])
    s6 = _gnn_pass(_layer_s_kernel, adjb, s5, W6, [(W6.shape[1], bf16)])

    bm = _block_m(n, 400)
    z_hat, z_adj, q = pl.pallas_call(
        functools.partial(_final_kernel, k=k),
        grid=(n // bm,),
        in_specs=[pl.BlockSpec((bm, n), lambda i: (i, 0)),
                  pl.BlockSpec(s6.shape, lambda i: (0, 0)),
                  pl.BlockSpec((bm, e), lambda i: (i, 0)),
                  pl.BlockSpec((n, e), lambda i: (0, 0)),
                  pl.BlockSpec((kpad, e), lambda i: (0, 0))],
        out_specs=[pl.BlockSpec((bm, W6.shape[1]), lambda i: (i, 0)),
                   pl.BlockSpec((bm, n), lambda i: (i, 0)),
                   pl.BlockSpec((bm, k), lambda i: (i, 0))],
        out_shape=[jax.ShapeDtypeStruct((n, W6.shape[1]), f32),
                   jax.ShapeDtypeStruct((n, n), f32),
                   jax.ShapeDtypeStruct((n, k), f32)],
        compiler_params=_PAR)(adjb, s6, z, z, c_pad)

    return (z_hat, z_adj, z, z_cluster, q)


# speedup vs baseline: 1.0417x; 1.0027x over previous
import functools

import jax
import jax.numpy as jnp
from jax.experimental import pallas as pl
from jax.experimental.pallas import tpu as pltpu

_V = 1.0
_PAR = pltpu.CompilerParams(dimension_semantics=("parallel",))


def _layer1_kernel(adj_ref, x_ref, w1_ref, w2_ref, snext_ref, adjb_ref):
    adjb = adj_ref[...].astype(jnp.bfloat16)
    adjb_ref[...] = adjb
    s1 = jnp.dot(x_ref[...], w1_ref[...],
                 preferred_element_type=jnp.float32).astype(jnp.bfloat16)
    h = jnp.dot(adjb, s1, preferred_element_type=jnp.float32)
    h = jnp.maximum(h, 0.0)
    snext_ref[...] = jnp.dot(h, w2_ref[...],
                             preferred_element_type=jnp.float32
                             ).astype(snext_ref.dtype)


def _layer_s_kernel(adj_ref, s_ref, w_ref, snext_ref):
    h = jnp.dot(adj_ref[...], s_ref[...], preferred_element_type=jnp.float32)
    h = jnp.maximum(h, 0.0)
    snext_ref[...] = jnp.dot(h, w_ref[...],
                             preferred_element_type=jnp.float32
                             ).astype(snext_ref.dtype)


def _layer_zs_kernel(adj_ref, s_ref, w_ref, z_ref, snext_ref):
    z = jnp.dot(adj_ref[...], s_ref[...], preferred_element_type=jnp.float32)
    z_ref[...] = z
    snext_ref[...] = jnp.dot(z, w_ref[...],
                             preferred_element_type=jnp.float32
                             ).astype(snext_ref.dtype)


def _layer_split_kernel(adj_ref, s_ref, w_ref, zc_ref, snext_ref, *, split):
    o = jnp.dot(adj_ref[...], s_ref[...], preferred_element_type=jnp.float32)
    o = jnp.maximum(o, 0.0)
    zc_ref[...] = o[:, :split]
    snext_ref[...] = jnp.dot(o[:, split:], w_ref[...],
                             preferred_element_type=jnp.float32
                             ).astype(snext_ref.dtype)


def _final_kernel(adj_ref, s_ref, zb_ref, z_ref, c_ref,
                  zhat_ref, zadj_ref, q_ref, *, k):
    o = jnp.dot(adj_ref[...], s_ref[...], preferred_element_type=jnp.float32)
    zhat_ref[...] = jnp.maximum(o, 0.0)
    zb = zb_ref[...]
    logits = jax.lax.dot_general(
        zb.astype(jnp.bfloat16), z_ref[...].astype(jnp.bfloat16),
        (((1,), (1,)), ((), ())),
        preferred_element_type=jnp.float32)
    zadj_ref[...] = jax.nn.sigmoid(logits)
    c = c_ref[...]
    cross = jax.lax.dot_general(
        zb, c, (((1,), (1,)), ((), ())),
        preferred_element_type=jnp.float32)
    d2 = (jnp.sum(zb * zb, axis=1, keepdims=True)
          + jnp.sum(c * c, axis=1)[None, :] - 2.0 * cross)
    qn = 1.0 / (1.0 + d2 / _V)
    qn = qn ** ((_V + 1.0) / 2.0)
    qn = qn[:, :k]
    q_ref[...] = qn / jnp.sum(qn, axis=1, keepdims=True)


def _block_m(n, target):
    for bm in (2000, 1000, 400, 200, 8):
        if bm <= target and n % bm == 0:
            return bm
    return n


def _gnn_pass(kernel_fn, adj, s, w, outs, bm_target=1000):
    n = adj.shape[0]
    bm = _block_m(n, bm_target)
    grid = (n // bm,)
    in_specs = [
        pl.BlockSpec((bm, n), lambda i: (i, 0)),
        pl.BlockSpec(s.shape, lambda i: (0, 0)),
    ]
    args = [adj, s]
    if w is not None:
        in_specs.append(pl.BlockSpec(w.shape, lambda i: (0, 0)))
        args.append(w)
    out_shape = [jax.ShapeDtypeStruct((n, fw), dt) for fw, dt in outs]
    out_specs = [pl.BlockSpec((bm, fw), lambda i: (i, 0)) for fw, _ in outs]
    res = pl.pallas_call(
        kernel_fn, grid=grid, in_specs=in_specs, out_specs=out_specs,
        out_shape=out_shape, compiler_params=_PAR)(*args)
    return res if len(res) > 1 else res[0]


def kernel(x, adj, W1, W2, W3, Wc, W4, W5, W6, cluster_layer):
    n = adj.shape[0]
    k, e = cluster_layer.shape
    kpad = max(8, -(-k // 8) * 8)
    c_pad = jnp.zeros((kpad, e), jnp.float32).at[:k].set(cluster_layer)
    f32, bf16 = jnp.float32, jnp.bfloat16

    bm1 = _block_m(n, 400)
    s2, adjb = pl.pallas_call(
        _layer1_kernel,
        grid=(n // bm1,),
        in_specs=[pl.BlockSpec((bm1, n), lambda i: (i, 0)),
                  pl.BlockSpec(x.shape, lambda i: (0, 0)),
                  pl.BlockSpec(W1.shape, lambda i: (0, 0)),
                  pl.BlockSpec(W2.shape, lambda i: (0, 0))],
        out_specs=[pl.BlockSpec((bm1, W2.shape[1]), lambda i: (i, 0)),
                   pl.BlockSpec((bm1, n), lambda i: (i, 0))],
        out_shape=[jax.ShapeDtypeStruct((n, W2.shape[1]), bf16),
                   jax.ShapeDtypeStruct((n, n), bf16)],
        compiler_params=_PAR)(adj, x, W1, W2)
    s3 = _gnn_pass(_layer_s_kernel, adjb, s2, W3, [(W3.shape[1], bf16)])
    w_cat = jnp.concatenate([Wc, W4], axis=1)
    z, s4 = _gnn_pass(_layer_zs_kernel, adjb, s3, w_cat,
                      [(e, f32), (w_cat.shape[1], bf16)])
    z_cluster, s5 = _gnn_pass(
        functools.partial(_layer_split_kernel, split=k),
        adjb, s4, W5, [(k, f32), (W5.shape[1], bf16)])
    s6 = _gnn_pass(_layer_s_kernel, adjb, s5, W6, [(W6.shape[1], bf16)])

    bm = _block_m(n, 400)
    z_hat, z_adj, q = pl.pallas_call(
        functools.partial(_final_kernel, k=k),
        grid=(n // bm,),
        in_specs=[pl.BlockSpec((bm, n), lambda i: (i, 0)),
                  pl.BlockSpec(s6.shape, lambda i: (0, 0)),
                  pl.BlockSpec((bm, e), lambda i: (i, 0)),
                  pl.BlockSpec((n, e), lambda i: (0, 0)),
                  pl.BlockSpec((kpad, e), lambda i: (0, 0))],
        out_specs=[pl.BlockSpec((bm, W6.shape[1]), lambda i: (i, 0)),
                   pl.BlockSpec((bm, n), lambda i: (i, 0)),
                   pl.BlockSpec((bm, k), lambda i: (i, 0))],
        out_shape=[jax.ShapeDtypeStruct((n, W6.shape[1]), f32),
                   jax.ShapeDtypeStruct((n, n), f32),
                   jax.ShapeDtypeStruct((n, k), f32)],
        compiler_params=_PAR)(adjb, s6, z, z, c_pad)

    return (z_hat, z_adj, z, z_cluster, q)
